# baseline (speedup 1.0000x reference)
import functools

import jax
import jax.numpy as jnp
from jax.experimental import pallas as pl
from jax.experimental.pallas import tpu as pltpu

N = 10000
E = 160000
A = 10
F = 128
NB = 8
RMAX = 5.0
P = 5
NL = 2
NM = 4
NOUT = 10
AVG = 16.0

EB = 4000
NBLK = 1000


def _silu(v):
    return v * (1.0 / (1.0 + jnp.exp(-v)))


def _edge_body(vec_ref, rw0_ref, rb0_ref, rw1_ref, rb1_ref, rw2_ref, rb2_ref,
               wtp0_ref, wtp1_ref, tpw0_ref, tpw1_ref, sh_ref):
    vec = vec_ref[...]
    l2 = jnp.sum(vec * vec, axis=1, keepdims=True) + 1e-12
    lengths = jnp.sqrt(l2)
    inv_len = 1.0 / lengths
    unit = vec * inv_len
    sq3 = jnp.sqrt(3.0)
    sh = jnp.concatenate(
        [jnp.ones_like(lengths), sq3 * unit[:, 1:2], sq3 * unit[:, 2:3],
         sq3 * unit[:, 0:1]], axis=1)
    sh_ref[...] = sh
    u = lengths / RMAX
    n = (jax.lax.broadcasted_iota(jnp.int32, (1, NB), 1) + 1).astype(jnp.float32)
    bessel = jnp.sqrt(2.0 / RMAX) * jnp.sin(n * jnp.pi * u) * inv_len
    env = (1.0 - ((P + 1.0) * (P + 2.0) / 2.0) * u ** P
           + P * (P + 2.0) * u ** (P + 1)
           - (P * (P + 1.0) / 2.0) * u ** (P + 2))
    env = jnp.where(u < 1.0, env, 0.0)
    ef = bessel * env
    r = _silu(jnp.dot(ef, rw0_ref[...], preferred_element_type=jnp.float32)
              + rb0_ref[...])
    r = _silu(jnp.dot(r, rw1_ref[...], preferred_element_type=jnp.float32)
              + rb1_ref[...])
    r = _silu(jnp.dot(r, rw2_ref[...], preferred_element_type=jnp.float32)
              + rb2_ref[...])
    tpw0_ref[...] = jnp.dot(r, wtp0_ref[...], preferred_element_type=jnp.float32)
    tpw1_ref[...] = jnp.dot(r, wtp1_ref[...], preferred_element_type=jnp.float32)


def _edge_stage(vec, rw0, rb0, rw1, rb1, rw2, rb2, wtp0, wtp1):
    grid = E // EB
    full = lambda i: (0, 0)
    return pl.pallas_call(
        _edge_body,
        grid=(grid,),
        in_specs=[
            pl.BlockSpec((EB, 4), lambda i: (i, 0)),
            pl.BlockSpec((NB, 64), full),
            pl.BlockSpec((1, 64), full),
            pl.BlockSpec((64, 64), full),
            pl.BlockSpec((1, 64), full),
            pl.BlockSpec((64, 64), full),
            pl.BlockSpec((1, 64), full),
            pl.BlockSpec((64, F), full),
            pl.BlockSpec((64, F), full),
        ],
        out_specs=[
            pl.BlockSpec((EB, F), lambda i: (i, 0)),
            pl.BlockSpec((EB, F), lambda i: (i, 0)),
            pl.BlockSpec((EB, 4), lambda i: (i, 0)),
        ],
        out_shape=[
            jax.ShapeDtypeStruct((E, F), jnp.float32),
            jax.ShapeDtypeStruct((E, F), jnp.float32),
            jax.ShapeDtypeStruct((E, 4), jnp.float32),
        ],
    )(vec, rw0, rb0, rw1, rb1, rw2, rb2, wtp0, wtp1)


def _node_body(with_sc, with_h, with_read,
               a0_ref, a1_ref, a2_ref, a3_ref, wp0_ref, wp0b_ref, wp1_ref,
               *rest):
    idx = 0
    rest = list(rest)
    if with_sc:
        f_refs = rest[0:4]
        x_ref = rest[4]
        wsc_ref = rest[5]
        rest = rest[6:]
    if with_h:
        wlin_ref = rest[0]
        rest = rest[1:]
    if with_read:
        wread_ref = rest[0]
        rest = rest[1:]
    out_refs = rest[0:4]
    rest = rest[4:]
    if with_h:
        h_ref = rest[0]
        rest = rest[1:]
    if with_read:
        prop_ref = rest[0]
        rest = rest[1:]

    a0 = a0_ref[...] * (1.0 / AVG)
    a1 = a1_ref[...] * (1.0 / AVG)
    a2 = a2_ref[...] * (1.0 / AVG)
    a3 = a3_ref[...] * (1.0 / AVG)
    sq = a0 * a0 + a1 * a1 + a2 * a2 + a3 * a3
    out0 = (jnp.dot(a0, wp0_ref[...], preferred_element_type=jnp.float32)
            + jnp.dot(sq, wp0b_ref[...], preferred_element_type=jnp.float32))
    outs = [out0]
    for am in (a1, a2, a3):
        outs.append(jnp.dot(am, wp1_ref[...], preferred_element_type=jnp.float32))
    if with_sc:
        x = x_ref[...]
        for m in range(NM):
            fm = f_refs[m][...]
            acc = outs[m]
            for a in range(A):
                za = fm * x[:, a:a + 1]
                acc = acc + jnp.dot(za, wsc_ref[a], preferred_element_type=jnp.float32)
            outs[m] = acc
    for m in range(NM):
        out_refs[m][...] = outs[m]
    if with_h:
        h_ref[...] = jnp.dot(outs[0], wlin_ref[...], preferred_element_type=jnp.float32)
    if with_read:
        prop_ref[...] = jnp.dot(outs[0], wread_ref[...], preferred_element_type=jnp.float32)


def _node_stage(a_list, wp0, wp0b, wp1, feats=None, x=None, wsc=None,
                wlin=None, wread=None):
    with_sc = feats is not None
    with_h = wlin is not None
    with_read = wread is not None
    grid = N // NBLK
    full = lambda i: (0, 0)
    nblk = lambda i: (i, 0)
    in_specs = [pl.BlockSpec((NBLK, F), nblk)] * 4 + [
        pl.BlockSpec((F, F), full),
        pl.BlockSpec((F, F), full),
        pl.BlockSpec((F, F), full),
    ]
    args = list(a_list) + [wp0, wp0b, wp1]
    if with_sc:
        in_specs += [pl.BlockSpec((NBLK, F), nblk)] * 4
        args += list(feats)
        in_specs += [pl.BlockSpec((NBLK, A), nblk),
                     pl.BlockSpec((A, F, F), lambda i: (0, 0, 0))]
        args += [x, wsc]
    if with_h:
        in_specs.append(pl.BlockSpec((F, F), full))
        args.append(wlin)
    if with_read:
        in_specs.append(pl.BlockSpec((F, NOUT), full))
        args.append(wread)
    out_specs = [pl.BlockSpec((NBLK, F), nblk)] * 4
    out_shape = [jax.ShapeDtypeStruct((N, F), jnp.float32)] * 4
    if with_h:
        out_specs.append(pl.BlockSpec((NBLK, F), nblk))
        out_shape.append(jax.ShapeDtypeStruct((N, F), jnp.float32))
    if with_read:
        out_specs.append(pl.BlockSpec((NBLK, NOUT), nblk))
        out_shape.append(jax.ShapeDtypeStruct((N, NOUT), jnp.float32))
    body = functools.partial(_node_body, with_sc, with_h, with_read)
    return pl.pallas_call(
        body,
        grid=(grid,),
        in_specs=in_specs,
        out_specs=out_specs,
        out_shape=out_shape,
    )(*args)


def _embed_body(x_ref, we_ref, wlin_ref, h0_ref, h_ref):
    h0 = jnp.dot(x_ref[...], we_ref[...], preferred_element_type=jnp.float32)
    h0_ref[...] = h0
    h_ref[...] = jnp.dot(h0, wlin_ref[...], preferred_element_type=jnp.float32)


def _embed_stage(x, w_embed, wlin0):
    grid = N // NBLK
    return pl.pallas_call(
        _embed_body,
        grid=(grid,),
        in_specs=[
            pl.BlockSpec((NBLK, A), lambda i: (i, 0)),
            pl.BlockSpec((A, F), lambda i: (0, 0)),
            pl.BlockSpec((F, F), lambda i: (0, 0)),
        ],
        out_specs=[
            pl.BlockSpec((NBLK, F), lambda i: (i, 0)),
            pl.BlockSpec((NBLK, F), lambda i: (i, 0)),
        ],
        out_shape=[
            jax.ShapeDtypeStruct((N, F), jnp.float32),
            jax.ShapeDtypeStruct((N, F), jnp.float32),
        ],
    )(x, w_embed, wlin0)


def kernel(pos_th, x, edge_index_th, W_embed, RW0, Rb0, RW1, Rb1, RW2, Rb2,
           Wtp, Wlin, Wp0, Wp0b, Wp1, Wsc, W_read):
    src = edge_index_th[0].astype(jnp.int32)
    dst = edge_index_th[1].astype(jnp.int32)

    vec3 = pos_th[dst] - pos_th[src]
    vec = jnp.pad(vec3, ((0, 0), (0, 1)))

    wtp = Wtp.reshape(2, 64, F, NL)
    h0, h = _embed_stage(x, W_embed, Wlin[0])

    feats_parts = None
    for i in range(2):
        tpw0, tpw1, sh = _edge_stage(
            vec, RW0[i], Rb0[i][None, :], RW1[i], Rb1[i][None, :],
            RW2[i], Rb2[i][None, :], wtp[i, :, :, 0], wtp[i, :, :, 1])
        h_src = h[src]
        u0 = tpw0 * h_src
        u1 = tpw1 * h_src
        a_list = [
            jax.ops.segment_sum(u0, dst, num_segments=N),
            jax.ops.segment_sum(u1 * sh[:, 1:2], dst, num_segments=N),
            jax.ops.segment_sum(u1 * sh[:, 2:3], dst, num_segments=N),
            jax.ops.segment_sum(u1 * sh[:, 3:4], dst, num_segments=N),
        ]
        if i == 0:
            feats_parts, h = None, None
            outs = _node_stage(a_list, Wp0[i], Wp0b[i], Wp1[i], wlin=Wlin[1])
            feats_parts = outs[0:4]
            h = outs[4]
        else:
            outs = _node_stage(a_list, Wp0[i], Wp0b[i], Wp1[i],
                               feats=feats_parts, x=x, wsc=Wsc,
                               wread=W_read)
            feats_parts = outs[0:4]
            prop = outs[4]

    feats = jnp.stack(feats_parts, axis=-1)
    return prop, feats.reshape(N, F * NM)


# speedup vs baseline: 2.9995x; 2.9995x over previous
import functools

import jax
import jax.numpy as jnp
from jax import lax
from jax.experimental import pallas as pl
from jax.experimental.pallas import tpu as pltpu
from jax.experimental.pallas import tpu_sc as plsc

N = 10000
E = 160000
A = 10
F = 128
NB = 8
RMAX = 5.0
P = 5
NL = 2
NM = 4
NOUT = 10
AVG = 16.0

NC = 2
NS = 16
NW = NC * NS

EB = 4000
NBLK = 1000

_GCHUNK = E // NW
_GK = 200
_SCHUNK = E // NS
_SK = 80
_SBLKS = (E // NS) // _SK
_NRSMALL = 624
_NRBIG = N - 15 * _NRSMALL


def _sc_mesh():
    return plsc.VectorSubcoreMesh(core_axis_name="c", subcore_axis_name="s")


def _vec_body(pos_hbm, src_hbm, dst_hbm, out_hbm, idxs, idxd, ps_v, pd_v, sem):
    c = lax.axis_index("c")
    s = lax.axis_index("s")
    wid = s * NC + c
    base = pl.multiple_of(wid * _GCHUNK, 8)
    pltpu.sync_copy(src_hbm.at[pl.ds(base, _GCHUNK)], idxs)
    pltpu.sync_copy(dst_hbm.at[pl.ds(base, _GCHUNK)], idxd)

    def blk(b, carry):
        e0 = pl.multiple_of(b * _GK, 8)
        pltpu.async_copy(pos_hbm.at[idxs.at[pl.ds(e0, _GK)]], ps_v, sem).wait()
        pltpu.async_copy(pos_hbm.at[idxd.at[pl.ds(e0, _GK)]], pd_v, sem).wait()

        def dif(k, carry2):
            pd_v[k] = pd_v[k] - ps_v[k]
            return carry2

        lax.fori_loop(0, _GK, dif, 0)
        pltpu.sync_copy(pd_v, out_hbm.at[pl.ds(pl.multiple_of(base + e0, 8), _GK)])
        return carry

    lax.fori_loop(0, _GCHUNK // _GK, blk, 0)


def _vec_stage(pos16, src, dst):
    kfn = pl.kernel(
        _vec_body,
        mesh=_sc_mesh(),
        compiler_params=pltpu.CompilerParams(use_tc_tiling_on_sc=False),
        out_type=jax.ShapeDtypeStruct((E, 16), jnp.float32),
        scratch_types=[
            pltpu.VMEM((_GCHUNK,), jnp.int32),
            pltpu.VMEM((_GCHUNK,), jnp.int32),
            pltpu.VMEM((_GK, 16), jnp.float32),
            pltpu.VMEM((_GK, 16), jnp.float32),
            pltpu.SemaphoreType.DMA,
        ],
    )
    return kfn(pos16, src, dst)


def _hgather_body(h_hbm, src_hbm, out_hbm, idx_all, rows0, rows1, sem0, sem1):
    c = lax.axis_index("c")
    s = lax.axis_index("s")
    wid = s * NC + c
    base = pl.multiple_of(wid * _GCHUNK, 8)
    pltpu.sync_copy(src_hbm.at[pl.ds(base, _GCHUNK)], idx_all)
    rows = (rows0, rows1)
    sems = (sem0, sem1)
    nblk = _GCHUNK // _GK

    def gath(i, ph):
        e0 = pl.multiple_of(i * _GK, 8)
        pltpu.async_copy(h_hbm.at[idx_all.at[pl.ds(e0, _GK)]], rows[ph], sems[ph])

    def gwait(ph):
        pltpu.make_async_copy(h_hbm.at[idx_all.at[pl.ds(0, _GK)]],
                              rows[ph], sems[ph]).wait()

    def wout(i, ph):
        e0 = pl.multiple_of(base + i * _GK, 8)
        pltpu.sync_copy(rows[ph], out_hbm.at[pl.ds(e0, _GK)])

    gath(0, 0)

    def pair(p, carry):
        i = p * 2
        gwait(0)
        gath(i + 1, 1)
        wout(i, 0)
        gwait(1)

        @pl.when(i + 2 < nblk)
        def _():
            gath(i + 2, 0)

        wout(i + 1, 1)
        return carry

    lax.fori_loop(0, nblk // 2, pair, 0)
    gwait(0)
    wout(nblk - 1, 0)


def _hgather_stage(h, src):
    kfn = pl.kernel(
        _hgather_body,
        mesh=_sc_mesh(),
        out_type=jax.ShapeDtypeStruct((E, F), jnp.float32),
        scratch_types=[
            pltpu.VMEM((_GCHUNK,), jnp.int32),
            pltpu.VMEM((_GK, F), jnp.float32),
            pltpu.VMEM((_GK, F), jnp.float32),
            pltpu.SemaphoreType.DMA,
            pltpu.SemaphoreType.DMA,
        ],
    )
    return kfn(h, src)


def _scatter_body(msg_hbm, dst3_hbm, zeros_hbm, out_hbm, acc, dst2, msg0, msg1,
                  sem0, sem1):
    c = lax.axis_index("c")
    s = lax.axis_index("s")
    base0 = pl.multiple_of(s * _SCHUNK, 8)
    pltpu.sync_copy(dst3_hbm.at[s], dst2)
    msgs = (msg0, msg1)
    sems = (sem0, sem1)
    for r in range(2):
        m = 2 * r + c

        @pl.when(s < NS - 1)
        def _():
            r0 = pl.multiple_of(s * _NRSMALL, 8)
            pltpu.sync_copy(zeros_hbm.at[pl.ds(r0, _NRSMALL)],
                            acc.at[pl.ds(r0, _NRSMALL)])

        @pl.when(s == NS - 1)
        def _():
            r0 = (NS - 1) * _NRSMALL
            pltpu.sync_copy(zeros_hbm.at[pl.ds(r0, _NRBIG)],
                            acc.at[pl.ds(r0, _NRBIG)])

        plsc.subcore_barrier()

        def gath(i, ph):
            e0 = pl.multiple_of(base0 + i * _SK, 8)
            pltpu.async_copy(msg_hbm.at[m, pl.ds(e0, _SK)], msgs[ph], sems[ph])

        def gwait(ph):
            pltpu.make_async_copy(msg_hbm.at[m, pl.ds(base0, _SK)],
                                  msgs[ph], sems[ph]).wait()

        def add(i, ph):
            pltpu.sync_copy(msgs[ph], acc.at[dst2.at[i]], add=True)

        gath(0, 0)

        def pair(p, carry):
            i = p * 2
            gwait(0)
            gath(i + 1, 1)
            add(i, 0)
            gwait(1)

            @pl.when(i + 2 < _SBLKS)
            def _():
                gath(i + 2, 0)

            add(i + 1, 1)
            return carry

        lax.fori_loop(0, _SBLKS // 2, pair, 0)
        gwait(0)
        add(_SBLKS - 1, 0)
        plsc.subcore_barrier()

        @pl.when(s < NS - 1)
        def _():
            r0 = pl.multiple_of(s * _NRSMALL, 8)
            pltpu.sync_copy(acc.at[pl.ds(r0, _NRSMALL)],
                            out_hbm.at[m, pl.ds(r0, _NRSMALL)])

        @pl.when(s == NS - 1)
        def _():
            r0 = (NS - 1) * _NRSMALL
            pltpu.sync_copy(acc.at[pl.ds(r0, _NRBIG)],
                            out_hbm.at[m, pl.ds(r0, _NRBIG)])

        plsc.subcore_barrier()


def _scatter_stage(msg, dst3, zeros_nf):
    kfn = pl.kernel(
        _scatter_body,
        mesh=_sc_mesh(),
        out_type=jax.ShapeDtypeStruct((NM, N, F), jnp.float32),
        scratch_types=[
            pltpu.VMEM_SHARED((N, F), jnp.float32),
            pltpu.VMEM((_SBLKS, _SK), jnp.int32),
            pltpu.VMEM((_SK, F), jnp.float32),
            pltpu.VMEM((_SK, F), jnp.float32),
            pltpu.SemaphoreType.DMA,
            pltpu.SemaphoreType.DMA,
        ],
    )
    return kfn(msg, dst3, zeros_nf)


def _edge_body(vec_ref, hsrc_ref, rw0_ref, rb0_ref, rw1_ref, rb1_ref,
               rw2_ref, rb2_ref, wtp0_ref, wtp1_ref, msg_ref):
    vec = vec_ref[...]
    l2 = jnp.sum(vec * vec, axis=1, keepdims=True) + 1e-12
    lengths = jnp.sqrt(l2)
    inv_len = 1.0 / lengths
    unit = vec * inv_len
    sq3 = jnp.sqrt(3.0)
    u = lengths / RMAX
    n = (lax.broadcasted_iota(jnp.int32, (1, NB), 1) + 1).astype(jnp.float32)
    bessel = jnp.sqrt(2.0 / RMAX) * jnp.sin(n * jnp.pi * u) * inv_len
    env = (1.0 - ((P + 1.0) * (P + 2.0) / 2.0) * u ** P
           + P * (P + 2.0) * u ** (P + 1)
           - (P * (P + 1.0) / 2.0) * u ** (P + 2))
    env = jnp.where(u < 1.0, env, 0.0)
    ef = bessel * env
    r = _silu(jnp.dot(ef, rw0_ref[...], preferred_element_type=jnp.float32)
              + rb0_ref[...])
    r = _silu(jnp.dot(r, rw1_ref[...], preferred_element_type=jnp.float32)
              + rb1_ref[...])
    r = _silu(jnp.dot(r, rw2_ref[...], preferred_element_type=jnp.float32)
              + rb2_ref[...])
    h = hsrc_ref[...]
    u0 = jnp.dot(r, wtp0_ref[...], preferred_element_type=jnp.float32) * h
    u1 = jnp.dot(r, wtp1_ref[...], preferred_element_type=jnp.float32) * h
    msg_ref[0] = u0
    msg_ref[1] = u1 * (sq3 * unit[:, 1:2])
    msg_ref[2] = u1 * (sq3 * unit[:, 2:3])
    msg_ref[3] = u1 * (sq3 * unit[:, 0:1])


def _silu(v):
    return v * (1.0 / (1.0 + jnp.exp(-v)))


def _edge_stage(vec, hsrc, rw0, rb0, rw1, rb1, rw2, rb2, wtp0, wtp1):
    grid = E // EB
    full = lambda i: (0, 0)
    return pl.pallas_call(
        _edge_body,
        grid=(grid,),
        in_specs=[
            pl.BlockSpec((EB, 16), lambda i: (i, 0)),
            pl.BlockSpec((EB, F), lambda i: (i, 0)),
            pl.BlockSpec((NB, 64), full),
            pl.BlockSpec((1, 64), full),
            pl.BlockSpec((64, 64), full),
            pl.BlockSpec((1, 64), full),
            pl.BlockSpec((64, 64), full),
            pl.BlockSpec((1, 64), full),
            pl.BlockSpec((64, F), full),
            pl.BlockSpec((64, F), full),
        ],
        out_specs=pl.BlockSpec((NM, EB, F), lambda i: (0, i, 0)),
        out_shape=jax.ShapeDtypeStruct((NM, E, F), jnp.float32),
    )(vec, hsrc, rw0, rb0, rw1, rb1, rw2, rb2, wtp0, wtp1)


def _node_body(with_sc, with_h, with_read,
               an_ref, wp0_ref, wp0b_ref, wp1_ref, *rest):
    rest = list(rest)
    if with_sc:
        f_refs = rest[0:4]
        x_ref = rest[4]
        wsc_ref = rest[5]
        rest = rest[6:]
    if with_h:
        wlin_ref = rest[0]
        rest = rest[1:]
    if with_read:
        wread_ref = rest[0]
        rest = rest[1:]
    out_refs = rest[0:4]
    rest = rest[4:]
    if with_h:
        h_ref = rest[0]
        rest = rest[1:]
    if with_read:
        prop_ref = rest[0]
        rest = rest[1:]

    a0 = an_ref[0] * (1.0 / AVG)
    a1 = an_ref[1] * (1.0 / AVG)
    a2 = an_ref[2] * (1.0 / AVG)
    a3 = an_ref[3] * (1.0 / AVG)
    sq = a0 * a0 + a1 * a1 + a2 * a2 + a3 * a3
    out0 = (jnp.dot(a0, wp0_ref[...], preferred_element_type=jnp.float32)
            + jnp.dot(sq, wp0b_ref[...], preferred_element_type=jnp.float32))
    outs = [out0]
    for am in (a1, a2, a3):
        outs.append(jnp.dot(am, wp1_ref[...], preferred_element_type=jnp.float32))
    if with_sc:
        x = x_ref[...]
        for m in range(NM):
            fm = f_refs[m][...]
            acc = outs[m]
            for a in range(A):
                za = fm * x[:, a:a + 1]
                acc = acc + jnp.dot(za, wsc_ref[a],
                                    preferred_element_type=jnp.float32)
            outs[m] = acc
    for m in range(NM):
        out_refs[m][...] = outs[m]
    if with_h:
        h_ref[...] = jnp.dot(outs[0], wlin_ref[...],
                             preferred_element_type=jnp.float32)
    if with_read:
        prop_ref[...] = jnp.dot(outs[0], wread_ref[...],
                                preferred_element_type=jnp.float32)


def _node_stage(an, wp0, wp0b, wp1, feats=None, x=None, wsc=None,
                wlin=None, wread=None):
    with_sc = feats is not None
    with_h = wlin is not None
    with_read = wread is not None
    grid = N // NBLK
    full = lambda i: (0, 0)
    nblk = lambda i: (i, 0)
    in_specs = [pl.BlockSpec((NM, NBLK, F), lambda i: (0, i, 0)),
                pl.BlockSpec((F, F), full),
                pl.BlockSpec((F, F), full),
                pl.BlockSpec((F, F), full)]
    args = [an, wp0, wp0b, wp1]
    if with_sc:
        in_specs += [pl.BlockSpec((NBLK, F), nblk)] * 4
        args += list(feats)
        in_specs += [pl.BlockSpec((NBLK, A), nblk),
                     pl.BlockSpec((A, F, F), lambda i: (0, 0, 0))]
        args += [x, wsc]
    if with_h:
        in_specs.append(pl.BlockSpec((F, F), full))
        args.append(wlin)
    if with_read:
        in_specs.append(pl.BlockSpec((F, NOUT), full))
        args.append(wread)
    out_specs = [pl.BlockSpec((NBLK, F), nblk)] * 4
    out_shape = [jax.ShapeDtypeStruct((N, F), jnp.float32)] * 4
    if with_h:
        out_specs.append(pl.BlockSpec((NBLK, F), nblk))
        out_shape.append(jax.ShapeDtypeStruct((N, F), jnp.float32))
    if with_read:
        out_specs.append(pl.BlockSpec((NBLK, NOUT), nblk))
        out_shape.append(jax.ShapeDtypeStruct((N, NOUT), jnp.float32))
    body = functools.partial(_node_body, with_sc, with_h, with_read)
    return pl.pallas_call(
        body,
        grid=(grid,),
        in_specs=in_specs,
        out_specs=out_specs,
        out_shape=out_shape,
    )(*args)


def _embed_body(x_ref, we_ref, wlin_ref, h0_ref, h_ref):
    h0 = jnp.dot(x_ref[...], we_ref[...], preferred_element_type=jnp.float32)
    h0_ref[...] = h0
    h_ref[...] = jnp.dot(h0, wlin_ref[...], preferred_element_type=jnp.float32)


def _embed_stage(x, w_embed, wlin0):
    grid = N // NBLK
    return pl.pallas_call(
        _embed_body,
        grid=(grid,),
        in_specs=[
            pl.BlockSpec((NBLK, A), lambda i: (i, 0)),
            pl.BlockSpec((A, F), lambda i: (0, 0)),
            pl.BlockSpec((F, F), lambda i: (0, 0)),
        ],
        out_specs=[
            pl.BlockSpec((NBLK, F), lambda i: (i, 0)),
            pl.BlockSpec((NBLK, F), lambda i: (i, 0)),
        ],
        out_shape=[
            jax.ShapeDtypeStruct((N, F), jnp.float32),
            jax.ShapeDtypeStruct((N, F), jnp.float32),
        ],
    )(x, w_embed, wlin0)


def kernel(pos_th, x, edge_index_th, W_embed, RW0, Rb0, RW1, Rb1, RW2, Rb2,
           Wtp, Wlin, Wp0, Wp0b, Wp1, Wsc, W_read):
    src = edge_index_th[0].astype(jnp.int32)
    dst = edge_index_th[1].astype(jnp.int32)
    dst3 = dst.reshape(NS, _SBLKS, _SK)
    pos16 = jnp.pad(pos_th, ((0, 0), (0, 13)))
    zeros_nf = jnp.zeros((N, F), jnp.float32)

    vec = _vec_stage(pos16, src, dst)
    wtp = Wtp.reshape(2, 64, F, NL)
    h0, h = _embed_stage(x, W_embed, Wlin[0])

    feats_parts = None
    prop = None
    for i in range(2):
        h_src = _hgather_stage(h, src)
        msg = _edge_stage(vec, h_src, RW0[i], Rb0[i][None, :], RW1[i],
                          Rb1[i][None, :], RW2[i], Rb2[i][None, :],
                          wtp[i, :, :, 0], wtp[i, :, :, 1])
        an = _scatter_stage(msg, dst3, zeros_nf)
        if i == 0:
            outs = _node_stage(an, Wp0[i], Wp0b[i], Wp1[i], wlin=Wlin[1])
            feats_parts = outs[0:4]
            h = outs[4]
        else:
            outs = _node_stage(an, Wp0[i], Wp0b[i], Wp1[i],
                               feats=feats_parts, x=x, wsc=Wsc,
                               wread=W_read)
            feats_parts = outs[0:4]
            prop = outs[4]

    feats = jnp.stack(feats_parts, axis=-1)
    return prop, feats.reshape(N, F * NM)
